# Initial kernel scaffold; baseline (speedup 1.0000x reference)
import jax
import jax.numpy as jnp
from jax.experimental import pallas as pl


def kernel(x, W, b, sim_matrix, temperature):
    raise NotImplementedError("write your pallas kernel here")


# speedup vs baseline: 1.0152x; 1.0152x over previous
import jax
import jax.numpy as jnp
from jax.experimental import pallas as pl

_BLK = 2048


def _gate_kernel(x_ref, wt_ref, b_ref, sim_ref, t_ref, o_ref):
    proj = jnp.dot(x_ref[...], wt_ref[...], preferred_element_type=jnp.float32)
    proj = proj + b_ref[...]
    norm = jnp.sqrt(jnp.sum(proj * proj, axis=-1, keepdims=True))
    projn = proj / jnp.maximum(norm, 1e-12)
    sim = sim_ref[...]
    cnorm = jnp.sqrt(jnp.sum(sim * sim, axis=0, keepdims=True))
    simn = (sim / jnp.maximum(cnorm, 1e-12)) * jnp.exp(t_ref[0, 0])
    o_ref[...] = jnp.dot(projn, simn, preferred_element_type=jnp.float32)


def kernel(x, W, b, sim_matrix, temperature):
    tokens, model_dim = x.shape
    proj_dim, _ = W.shape
    num_experts = sim_matrix.shape[1]
    wt = W.T
    b2 = b.reshape(1, proj_dim)
    t2 = temperature.reshape(1, 1)
    grid = (tokens // _BLK,)
    return pl.pallas_call(
        _gate_kernel,
        grid=grid,
        in_specs=[
            pl.BlockSpec((_BLK, model_dim), lambda i: (i, 0)),
            pl.BlockSpec((model_dim, proj_dim), lambda i: (0, 0)),
            pl.BlockSpec((1, proj_dim), lambda i: (0, 0)),
            pl.BlockSpec((proj_dim, num_experts), lambda i: (0, 0)),
            pl.BlockSpec((1, 1), lambda i: (0, 0)),
        ],
        out_specs=pl.BlockSpec((_BLK, num_experts), lambda i: (i, 0)),
        out_shape=jax.ShapeDtypeStruct((tokens, num_experts), jnp.float32),
    )(x, wt, b2, sim_matrix, t2)
